# Initial kernel scaffold; baseline (speedup 1.0000x reference)
import jax
import jax.numpy as jnp
from jax.experimental import pallas as pl


def kernel(x, w1, b1, w2, b2, fc1_w, fc1_b, fc2_w, fc2_b):
    raise NotImplementedError("write your pallas kernel here")


# speedup vs baseline: 684.1542x; 684.1542x over previous
import jax
import jax.numpy as jnp
from jax.experimental import pallas as pl
from jax.experimental.pallas import tpu as pltpu


_C1_BLK = 2944
_C2_BLK = 896


def _net_kernel(x_ref, w1_ref, b1_ref, w2_ref, b2_ref, f1w_ref, f1b_ref,
                f2w_ref, f2b_ref, o_ref):
    xb = x_ref[...].astype(jnp.bfloat16)

    m1 = jnp.dot(xb, w1_ref[:, :_C1_BLK],
                 preferred_element_type=jnp.float32)
    for p in range(1, 4):
        z = jnp.dot(xb, w1_ref[:, p * _C1_BLK:(p + 1) * _C1_BLK],
                    preferred_element_type=jnp.float32)
        m1 = jnp.maximum(m1, z)
    a1 = jnp.maximum(m1 + b1_ref[...], 0.0).astype(jnp.bfloat16)

    m2 = jnp.dot(a1, w2_ref[:, :_C2_BLK],
                 preferred_element_type=jnp.float32)
    for p in range(1, 4):
        z = jnp.dot(a1, w2_ref[:, p * _C2_BLK:(p + 1) * _C2_BLK],
                    preferred_element_type=jnp.float32)
        m2 = jnp.maximum(m2, z)
    a2 = jnp.maximum(m2 + b2_ref[...], 0.0).astype(jnp.bfloat16)

    h = jnp.dot(a2, f1w_ref[...], preferred_element_type=jnp.float32)
    h = jnp.maximum(h + f1b_ref[...], 0.0).astype(jnp.bfloat16)
    y = jnp.dot(h, f2w_ref[...], preferred_element_type=jnp.float32)
    y = y + f2b_ref[...]
    m = jnp.max(y, axis=-1, keepdims=True)
    s = y - m
    lse = jnp.log(jnp.sum(jnp.exp(s), axis=-1, keepdims=True))
    o_ref[...] = (s - lse).astype(o_ref.dtype)


def _spread_conv1(w1):
    w15 = w1.reshape(5, 5, 20)
    h = jnp.arange(28)[None, :, None]
    p = jnp.arange(12)[:, None, None]
    k = jnp.arange(5)[None, None, :]
    sel = [(h == 2 * p + a + k).astype(jnp.float32) for a in (0, 1)]
    blocks = []
    for a in (0, 1):
        for b in (0, 1):
            t = jnp.einsum('phk,qwm,kmo->hwpqo', sel[a], sel[b], w15)
            blocks.append(jnp.pad(t.reshape(784, 2880), ((0, 0), (0, 64))))
    return jnp.concatenate(blocks, axis=1).astype(jnp.bfloat16)


def _spread_conv2(w2):
    w25 = w2.reshape(5, 5, 20, 50)
    pq = jnp.arange(12)[None, :, None]
    y = jnp.arange(4)[:, None, None]
    k = jnp.arange(5)[None, None, :]
    sel = [(pq == 2 * y + a + k).astype(jnp.float32) for a in (0, 1)]
    blocks = []
    for a in (0, 1):
        for b in (0, 1):
            u = jnp.einsum('ypk,xqm,kmio->pqiyxo', sel[a], sel[b], w25)
            blocks.append(jnp.pad(u.reshape(2880, 800), ((0, 64), (0, 96))))
    return jnp.concatenate(blocks, axis=1).astype(jnp.bfloat16)


def kernel(x, w1, b1, w2, b2, fc1_w, fc1_b, fc2_w, fc2_b):
    B = x.shape[0]
    xf = x.reshape(B, 784)

    W1 = _spread_conv1(w1)
    b1row = jnp.pad(jnp.tile(b1, (1, 144)), ((0, 0), (0, 64)))
    W2 = _spread_conv2(w2)
    b2row = jnp.pad(jnp.tile(b2, (1, 16)), ((0, 0), (0, 96)))
    f1w = jnp.pad(fc1_w, ((0, 96), (0, 0))).astype(jnp.bfloat16)
    f2w = fc2_w.astype(jnp.bfloat16)

    TB = next((t for t in (128, 64, 32, 16, 8) if B % t == 0), B)
    row_spec = lambda shape: pl.BlockSpec(shape, lambda i: (i, 0))
    full_spec = lambda shape: pl.BlockSpec(shape, lambda i: (0, 0))

    return pl.pallas_call(
        _net_kernel,
        out_shape=jax.ShapeDtypeStruct((B, 10), jnp.float32),
        grid=(B // TB,),
        in_specs=[
            row_spec((TB, 784)),
            full_spec(W1.shape),
            full_spec(b1row.shape),
            full_spec(W2.shape),
            full_spec(b2row.shape),
            full_spec(f1w.shape),
            full_spec(fc1_b.shape),
            full_spec(f2w.shape),
            full_spec(fc2_b.shape),
        ],
        out_specs=row_spec((TB, 10)),
        compiler_params=pltpu.CompilerParams(
            dimension_semantics=("parallel",),
            vmem_limit_bytes=100 * 1024 * 1024,
        ),
    )(xf, W1, b1row, W2, b2row, f1w, fc1_b, f2w, fc2_b)


# speedup vs baseline: 1791.0631x; 2.6179x over previous
import jax
import jax.numpy as jnp
from jax.experimental import pallas as pl
from jax.experimental.pallas import tpu as pltpu


def _net_kernel(x_ref, w1_ref, b1_ref, w2_ref, b2_ref, f1w_ref, f1b_ref,
                f2w_ref, f2b_ref, o_ref):
    xb = x_ref[...].astype(jnp.bfloat16)

    w1 = w1_ref[...]
    a1_bands = []
    for g in range(6):
        z = jnp.dot(xb[:, 112 * g:112 * g + 224], w1,
                    preferred_element_type=jnp.float32)
        m = jnp.maximum(jnp.maximum(z[:, :512], z[:, 512:1024]),
                        jnp.maximum(z[:, 1024:1536], z[:, 1536:]))
        a1_bands.append(
            jnp.maximum(m + b1_ref[...], 0.0).astype(jnp.bfloat16))
    a1 = jnp.concatenate(a1_bands, axis=1)

    w2 = w2_ref[...]
    a2_bands = []
    for y in range(4):
        z = jnp.dot(a1[:, 512 * y:512 * y + 1536], w2,
                    preferred_element_type=jnp.float32)
        m = jnp.maximum(jnp.maximum(z[:, :256], z[:, 256:512]),
                        jnp.maximum(z[:, 512:768], z[:, 768:]))
        a2_bands.append(
            jnp.maximum(m + b2_ref[...], 0.0).astype(jnp.bfloat16))
    a2 = jnp.concatenate(a2_bands, axis=1)

    h = jnp.dot(a2, f1w_ref[...], preferred_element_type=jnp.float32)
    h = jnp.maximum(h + f1b_ref[...], 0.0).astype(jnp.bfloat16)
    y = jnp.dot(h, f2w_ref[...], preferred_element_type=jnp.float32)
    y = y + f2b_ref[...]
    m = jnp.max(y, axis=-1, keepdims=True)
    s = y - m
    lse = jnp.log(jnp.sum(jnp.exp(s), axis=-1, keepdims=True))
    o_ref[...] = (s - lse).astype(o_ref.dtype)


def _spread_conv1(w1):
    w15 = w1.reshape(5, 5, 20)
    h = jnp.arange(8)[None, :, None]
    p = jnp.arange(2)[:, None, None]
    k = jnp.arange(5)[None, None, :]
    sel_h = [(h == 2 * p + a + k).astype(jnp.float32) for a in (0, 1)]
    w = jnp.arange(28)[None, :, None]
    q = jnp.arange(12)[:, None, None]
    sel_w = [(w == 2 * q + b + k).astype(jnp.float32) for b in (0, 1)]
    blocks = []
    for a in (0, 1):
        for b in (0, 1):
            t = jnp.einsum('phk,qwm,kmo->hwpqo', sel_h[a], sel_w[b], w15)
            blocks.append(jnp.pad(t.reshape(224, 480), ((0, 0), (0, 32))))
    return jnp.concatenate(blocks, axis=1).astype(jnp.bfloat16)


def _spread_conv2(w2):
    w25 = w2.reshape(5, 5, 20, 50)
    q = jnp.arange(12)[None, :, None]
    x = jnp.arange(4)[:, None, None]
    k = jnp.arange(5)[None, None, :]
    sel_w = [(q == 2 * x + b + k).astype(jnp.float32) for b in (0, 1)]
    blocks = []
    for a in (0, 1):
        wa = jnp.pad(w25, ((a, 1 - a), (0, 0), (0, 0), (0, 0)))
        for b in (0, 1):
            t = jnp.einsum('xqm,kmio->kqixo', sel_w[b], wa)
            t = t.reshape(3, 480, 200)
            t = jnp.pad(t, ((0, 0), (0, 32), (0, 0))).reshape(1536, 200)
            blocks.append(jnp.pad(t, ((0, 0), (0, 56))))
    return jnp.concatenate(blocks, axis=1).astype(jnp.bfloat16)


def kernel(x, w1, b1, w2, b2, fc1_w, fc1_b, fc2_w, fc2_b):
    B = x.shape[0]
    xf = x.reshape(B, 784)

    W1 = _spread_conv1(w1)
    b1row = jnp.pad(jnp.tile(b1, (1, 24)), ((0, 0), (0, 32)))
    W2 = _spread_conv2(w2)
    b2row = jnp.pad(jnp.tile(b2, (1, 4)), ((0, 0), (0, 56)))
    f1w = jnp.pad(fc1_w.reshape(4, 200, 256),
                  ((0, 0), (0, 56), (0, 0))).reshape(1024, 256)
    f1w = f1w.astype(jnp.bfloat16)
    f2w = fc2_w.astype(jnp.bfloat16)

    TB = next((t for t in (256, 128, 64, 32, 16, 8) if B % t == 0), B)
    row_spec = lambda shape: pl.BlockSpec(shape, lambda i: (i, 0))
    full_spec = lambda shape: pl.BlockSpec(shape, lambda i: (0, 0))

    return pl.pallas_call(
        _net_kernel,
        out_shape=jax.ShapeDtypeStruct((B, 10), jnp.float32),
        grid=(B // TB,),
        in_specs=[
            row_spec((TB, 784)),
            full_spec(W1.shape),
            full_spec(b1row.shape),
            full_spec(W2.shape),
            full_spec(b2row.shape),
            full_spec(f1w.shape),
            full_spec(fc1_b.shape),
            full_spec(f2w.shape),
            full_spec(fc2_b.shape),
        ],
        out_specs=row_spec((TB, 10)),
        compiler_params=pltpu.CompilerParams(
            dimension_semantics=("parallel",),
            vmem_limit_bytes=100 * 1024 * 1024,
        ),
    )(xf, W1, b1row, W2, b2row, f1w, fc1_b, f2w, fc2_b)


# speedup vs baseline: 2048.8464x; 1.1439x over previous
import numpy as np

import jax
import jax.numpy as jnp
from jax.experimental import pallas as pl
from jax.experimental.pallas import tpu as pltpu


def _conv1_selector():
    h = np.arange(8)
    w = np.arange(28)
    p = np.arange(2)
    q = np.arange(12)
    k = np.arange(5)
    mats = []
    for a in (0, 1):
        A = (h[:, None, None] == 2 * p[None, :, None] + a + k[None, None, :])
        for b in (0, 1):
            Bm = (w[:, None, None] == 2 * q[None, :, None] + b + k[None, None, :])
            m = np.einsum('hpk,wqm->hwpqkm', A, Bm).reshape(5376, 25)
            mats.append(m)
    return np.stack(mats).astype(np.float32)


def _conv2_selector():
    ph = np.arange(6)
    q = np.arange(12)
    x = np.arange(4)
    k = np.arange(5)
    mats = []
    for a in (0, 1):
        C = (k[None, :] == ph[:, None] - a)
        for b in (0, 1):
            D = (q[:, None, None] == 2 * x[None, :, None] + b + k[None, None, :])
            m = np.einsum('pk,qxm->pqxkm', C, D).reshape(288, 25)
            mats.append(m)
    return np.stack(mats).astype(np.float32)


_SEL1 = _conv1_selector()
_SEL2 = _conv2_selector()


def _net_kernel(x_ref, w1_ref, b1_ref, w2_ref, b2_ref, f1w_ref, f1b_ref,
                f2w_ref, f2b_ref, o_ref):
    tb = x_ref.shape[0]
    xb = x_ref[...]

    x1 = jnp.concatenate([xb[:, 128 * g:128 * g + 256] for g in range(6)],
                         axis=0)
    z1 = jnp.dot(x1, w1_ref[...], preferred_element_type=jnp.float32)
    a1_bands = []
    for g in range(6):
        z = z1[tb * g:tb * (g + 1)]
        m = jnp.maximum(jnp.maximum(z[:, :512], z[:, 512:1024]),
                        jnp.maximum(z[:, 1024:1536], z[:, 1536:]))
        a1_bands.append(
            jnp.maximum(m + b1_ref[...], 0.0).astype(jnp.bfloat16))
    a1 = jnp.concatenate(a1_bands, axis=1)

    l2 = jnp.concatenate([a1[:, 512 * y:512 * y + 1536] for y in range(4)],
                         axis=0)
    z2 = jnp.dot(l2, w2_ref[...], preferred_element_type=jnp.float32)
    a2_bands = []
    for y in range(4):
        z = z2[tb * y:tb * (y + 1)]
        m = jnp.maximum(jnp.maximum(z[:, :256], z[:, 256:512]),
                        jnp.maximum(z[:, 512:768], z[:, 768:]))
        a2_bands.append(
            jnp.maximum(m + b2_ref[...], 0.0).astype(jnp.bfloat16))
    a2 = jnp.concatenate(a2_bands, axis=1)

    h = jnp.dot(a2, f1w_ref[...], preferred_element_type=jnp.float32)
    h = jnp.maximum(h + f1b_ref[...], 0.0).astype(jnp.bfloat16)
    y = jnp.dot(h, f2w_ref[...], preferred_element_type=jnp.float32)
    y = y + f2b_ref[...]
    m = jnp.max(y, axis=-1, keepdims=True)
    s = y - m
    lse = jnp.log(jnp.sum(jnp.exp(s), axis=-1, keepdims=True))
    o_ref[...] = (s - lse).astype(o_ref.dtype)


def _spread_conv1(w1):
    t = jnp.dot(jnp.asarray(_SEL1).reshape(4 * 5376, 25), w1)
    t = t.reshape(4, 8, 28, 2, 12, 20)
    t = jnp.transpose(t, (1, 2, 0, 3, 4, 5)).reshape(8, 28, 4, 480)
    t = jnp.pad(t, ((0, 0), (0, 4), (0, 0), (0, 32)))
    return t.reshape(256, 2048).astype(jnp.bfloat16)


def _spread_conv2(w2):
    w2r = w2.reshape(25, 1000)
    t = jnp.dot(jnp.asarray(_SEL2).reshape(4 * 288, 25), w2r)
    t = t.reshape(4, 6, 12, 4, 20, 50)
    t = jnp.transpose(t, (0, 1, 2, 4, 3, 5))
    t = t.reshape(4, 3, 480, 200)
    t = jnp.pad(t, ((0, 0), (0, 0), (0, 32), (0, 56)))
    t = jnp.transpose(t.reshape(4, 1536, 256), (1, 0, 2))
    return t.reshape(1536, 1024).astype(jnp.bfloat16)


def kernel(x, w1, b1, w2, b2, fc1_w, fc1_b, fc2_w, fc2_b):
    B = x.shape[0]
    xp = jnp.pad(x.reshape(B, 28, 28), ((0, 0), (0, 0), (0, 4)))
    xp = xp.reshape(B, 896).astype(jnp.bfloat16)

    W1 = _spread_conv1(w1)
    b1row = jnp.pad(jnp.tile(b1, (1, 24)), ((0, 0), (0, 32)))
    W2 = _spread_conv2(w2)
    b2row = jnp.pad(jnp.tile(b2, (1, 4)), ((0, 0), (0, 56)))
    f1w = jnp.pad(fc1_w.reshape(4, 200, 256),
                  ((0, 0), (0, 56), (0, 0))).reshape(1024, 256)
    f1w = f1w.astype(jnp.bfloat16)
    f2w = fc2_w.astype(jnp.bfloat16)

    TB = next((t for t in (256, 128, 64, 32, 16, 8) if B % t == 0), B)
    row_spec = lambda shape: pl.BlockSpec(shape, lambda i: (i, 0))
    full_spec = lambda shape: pl.BlockSpec(shape, lambda i: (0, 0))

    return pl.pallas_call(
        _net_kernel,
        out_shape=jax.ShapeDtypeStruct((B, 10), jnp.float32),
        grid=(B // TB,),
        in_specs=[
            row_spec((TB, 896)),
            full_spec(W1.shape),
            full_spec(b1row.shape),
            full_spec(W2.shape),
            full_spec(b2row.shape),
            full_spec(f1w.shape),
            full_spec(fc1_b.shape),
            full_spec(f2w.shape),
            full_spec(fc2_b.shape),
        ],
        out_specs=row_spec((TB, 10)),
        compiler_params=pltpu.CompilerParams(
            dimension_semantics=("parallel",),
            vmem_limit_bytes=100 * 1024 * 1024,
        ),
    )(xp, W1, b1row, W2, b2row, f1w, fc1_b, f2w, fc2_b)
